# baseline (speedup 1.0000x reference)
import functools

import jax
import jax.numpy as jnp
from jax import lax
from jax.experimental import pallas as pl
from jax.experimental.pallas import tpu as pltpu
from jax.experimental.pallas import tpu_sc as plsc

USER_N = 1000000
ITEM_N = 100000
BATCH = 16384
DIM = 16
NUM_CORES = 2
NUM_SUBCORES = 16
NUM_WORKERS = NUM_CORES * NUM_SUBCORES
BPW = BATCH // NUM_WORKERS
LANES = 16
GROUPS = BPW // LANES

_mesh = plsc.VectorSubcoreMesh(core_axis_name="c", subcore_axis_name="s")


@functools.partial(
    pl.kernel,
    mesh=_mesh,
    compiler_params=pltpu.CompilerParams(needs_layout_passes=False,
                                         use_tc_tiling_on_sc=False),
    out_type=jax.ShapeDtypeStruct((BATCH,), jnp.float32),
    scratch_types=[
        pltpu.VMEM((BPW,), jnp.int32),
        pltpu.VMEM((BPW,), jnp.int32),
        pltpu.VMEM((DIM, BPW), jnp.int32),
        pltpu.VMEM((DIM, BPW), jnp.int32),
        pltpu.VMEM((DIM, BPW), jnp.float32),
        pltpu.VMEM((DIM, BPW), jnp.float32),
        pltpu.VMEM((BPW,), jnp.float32),
        pltpu.VMEM((BPW,), jnp.float32),
        pltpu.SemaphoreType.DMA,
    ],
)
def _mirt_sc(user_hbm, item_hbm, theta_f_hbm, a_f_hbm, b_f_hbm, out_hbm,
             uidx_v, iidx_v, tidx_v, aidx_v, th_v, a_v, b_v, out_v, sem):
    wid = lax.axis_index("s") * NUM_CORES + lax.axis_index("c")
    base = wid * BPW

    pltpu.sync_copy(user_hbm.at[pl.ds(base, BPW)], uidx_v)
    pltpu.sync_copy(item_hbm.at[pl.ds(base, BPW)], iidx_v)

    def idx_body(g, carry):
        sl = pl.ds(g * LANES, LANES)
        u = uidx_v[sl]
        it = iidx_v[sl]
        for d in range(DIM):
            tidx_v[d, sl] = u + (d * USER_N)
            aidx_v[d, sl] = it + (d * ITEM_N)
        return carry

    lax.fori_loop(0, GROUPS, idx_body, 0)

    copies = [pltpu.async_copy(b_f_hbm.at[iidx_v], b_v, sem)]
    for d in range(DIM):
        copies.append(
            pltpu.async_copy(theta_f_hbm.at[tidx_v.at[d]], th_v.at[d], sem))
        copies.append(
            pltpu.async_copy(a_f_hbm.at[aidx_v.at[d]], a_v.at[d], sem))
    for c in copies:
        c.wait()

    def group_body(g, carry):
        sl = pl.ds(g * LANES, LANES)
        acc = jnp.zeros((LANES,), jnp.float32)
        for d in range(DIM):
            acc = acc + th_v[d, sl] * a_v[d, sl]
        out_v[sl] = 1.0 / (1.0 + jnp.exp(b_v[sl] - acc))
        return carry

    lax.fori_loop(0, GROUPS, group_body, 0)

    pltpu.sync_copy(out_v, out_hbm.at[pl.ds(base, BPW)])


def kernel(user, item, theta_w, a_w, b_w):
    return _mirt_sc(user.astype(jnp.int32), item.astype(jnp.int32),
                    theta_w.T.reshape(-1), a_w.T.reshape(-1),
                    b_w.reshape(-1))


# speedup vs baseline: 1.0062x; 1.0062x over previous
import functools

import jax
import jax.numpy as jnp
from jax import lax
from jax.experimental import pallas as pl
from jax.experimental.pallas import tpu as pltpu
from jax.experimental.pallas import tpu_sc as plsc

USER_N = 1000000
ITEM_N = 100000
BATCH = 16384
DIM = 16
NUM_CORES = 2
NUM_SUBCORES = 16
NUM_WORKERS = NUM_CORES * NUM_SUBCORES
BPW = BATCH // NUM_WORKERS
LANES = 16
GROUPS = BPW // LANES

_mesh = plsc.VectorSubcoreMesh(core_axis_name="c", subcore_axis_name="s")


@functools.partial(
    pl.kernel,
    mesh=_mesh,
    compiler_params=pltpu.CompilerParams(needs_layout_passes=False,
                                         use_tc_tiling_on_sc=False),
    out_type=jax.ShapeDtypeStruct((BATCH,), jnp.float32),
    scratch_types=[
        pltpu.VMEM((BPW,), jnp.int32),
        pltpu.VMEM((BPW,), jnp.int32),
        pltpu.VMEM((DIM, BPW), jnp.float32),
        pltpu.VMEM((DIM, BPW), jnp.float32),
        pltpu.VMEM((BPW,), jnp.float32),
        pltpu.VMEM((BPW,), jnp.float32),
        pltpu.SemaphoreType.DMA,
    ],
)
def _mirt_sc(user_hbm, item_hbm, theta_t_hbm, a_t_hbm, b_t_hbm, out_hbm,
             uidx_v, iidx_v, th_v, a_v, b_v, out_v, sem):
    wid = lax.axis_index("s") * NUM_CORES + lax.axis_index("c")
    base = wid * BPW

    pltpu.sync_copy(user_hbm.at[pl.ds(base, BPW)], uidx_v)
    pltpu.sync_copy(item_hbm.at[pl.ds(base, BPW)], iidx_v)

    copies = [pltpu.async_copy(b_t_hbm.at[0].at[iidx_v], b_v, sem)]
    for d in range(DIM):
        copies.append(
            pltpu.async_copy(theta_t_hbm.at[d].at[uidx_v], th_v.at[d], sem))
        copies.append(
            pltpu.async_copy(a_t_hbm.at[d].at[iidx_v], a_v.at[d], sem))
    for c in copies:
        c.wait()

    def group_body(g, carry):
        sl = pl.ds(g * LANES, LANES)
        acc = jnp.zeros((LANES,), jnp.float32)
        for d in range(DIM):
            acc = acc + th_v[d, sl] * a_v[d, sl]
        out_v[sl] = 1.0 / (1.0 + jnp.exp(b_v[sl] - acc))
        return carry

    lax.fori_loop(0, GROUPS, group_body, 0)

    pltpu.sync_copy(out_v, out_hbm.at[pl.ds(base, BPW)])


def kernel(user, item, theta_w, a_w, b_w):
    return _mirt_sc(user.astype(jnp.int32), item.astype(jnp.int32),
                    theta_w.T, a_w.T, b_w.T)


# speedup vs baseline: 13.4448x; 13.3614x over previous
import functools

import jax
import jax.numpy as jnp
from jax import lax
from jax.experimental.layout import Layout, with_layout_constraint
from jax.experimental import pallas as pl
from jax.experimental.pallas import tpu as pltpu
from jax.experimental.pallas import tpu_sc as plsc

USER_N = 1000000
ITEM_N = 100000
BATCH = 16384
DIM = 16
NUM_CORES = 2
NUM_SUBCORES = 16
NUM_WORKERS = NUM_CORES * NUM_SUBCORES
BPW = BATCH // NUM_WORKERS
LANES = 16
GROUPS = BPW // LANES

USER_ROW_STRIDE = -(-USER_N // 128) * 1024
ITEM_ROW_STRIDE = -(-ITEM_N // 128) * 1024

_mesh = plsc.VectorSubcoreMesh(core_axis_name="c", subcore_axis_name="s")


@functools.partial(
    pl.kernel,
    mesh=_mesh,
    compiler_params=pltpu.CompilerParams(needs_layout_passes=False,
                                         use_tc_tiling_on_sc=False),
    out_type=jax.ShapeDtypeStruct((BATCH,), jnp.float32),
    scratch_types=[
        pltpu.VMEM((BPW,), jnp.int32),
        pltpu.VMEM((BPW,), jnp.int32),
        pltpu.VMEM((DIM, BPW), jnp.int32),
        pltpu.VMEM((DIM, BPW), jnp.int32),
        pltpu.VMEM((DIM, BPW), jnp.float32),
        pltpu.VMEM((DIM, BPW), jnp.float32),
        pltpu.VMEM((BPW,), jnp.float32),
        pltpu.VMEM((BPW,), jnp.float32),
        pltpu.SemaphoreType.DMA,
    ],
)
def _mirt_sc(user_hbm, item_hbm, theta_t_hbm, a_t_hbm, b_hbm, out_hbm,
             uidx_v, iidx_v, tidx_v, aidx_v, th_v, a_v, b_v, out_v, sem):
    wid = lax.axis_index("s") * NUM_CORES + lax.axis_index("c")
    base = wid * BPW

    pltpu.sync_copy(user_hbm.at[pl.ds(base, BPW)], uidx_v)
    pltpu.sync_copy(item_hbm.at[pl.ds(base, BPW)], iidx_v)

    def idx_body(g, carry):
        sl = pl.ds(g * LANES, LANES)
        u = uidx_v[sl]
        it = iidx_v[sl]
        pu = ((u >> 7) << 10) + (u & 127)
        pi = ((it >> 7) << 10) + (it & 127)
        for d in range(DIM):
            tidx_v[d, sl] = pu + ((d // 8) * USER_ROW_STRIDE + (d % 8) * 128)
            aidx_v[d, sl] = pi + ((d // 8) * ITEM_ROW_STRIDE + (d % 8) * 128)
        return carry

    lax.fori_loop(0, GROUPS, idx_body, 0)

    copies = [pltpu.async_copy(b_hbm.at[iidx_v], b_v, sem)]
    for d in range(DIM):
        copies.append(pltpu.async_copy(
            theta_t_hbm.at[0].at[tidx_v.at[d]], th_v.at[d], sem))
        copies.append(pltpu.async_copy(
            a_t_hbm.at[0].at[aidx_v.at[d]], a_v.at[d], sem))
    for c in copies:
        c.wait()

    def group_body(g, carry):
        sl = pl.ds(g * LANES, LANES)
        acc = jnp.zeros((LANES,), jnp.float32)
        for d in range(DIM):
            acc = acc + th_v[d, sl] * a_v[d, sl]
        out_v[sl] = 1.0 / (1.0 + jnp.exp(b_v[sl] - acc))
        return carry

    lax.fori_loop(0, GROUPS, group_body, 0)

    pltpu.sync_copy(out_v, out_hbm.at[pl.ds(base, BPW)])


def kernel(user, item, theta_w, a_w, b_w):
    fmt2 = Layout(major_to_minor=(0, 1), tiling=((8, 128),))
    theta_t = with_layout_constraint(theta_w.T, fmt2)
    a_t = with_layout_constraint(a_w.T, fmt2)
    return _mirt_sc(user.astype(jnp.int32), item.astype(jnp.int32),
                    theta_t, a_t, jnp.reshape(b_w, (-1,)))
